# baseline (speedup 1.0000x reference)
import functools

import jax
import jax.numpy as jnp
from jax import lax
from jax.experimental import pallas as pl
from jax.experimental.pallas import tpu as pltpu
from jax.experimental.pallas import tpu_sc as plsc

GENE_NUM = 100000
D = 64
BATCH = 4096
SEQ = 200
EPS = 1e-5

NC = 2
NS = 16
NW = NC * NS
NTOT = BATCH * SEQ
PER_W = NTOT // NW
BAT_W = BATCH // NW
CW = SEQ // 2
NCH = PER_W // CW
NB = 4
NG = NCH // NB


def _ln_body(t_ref, g_ref, b_ref, o_ref):
    t = t_ref[...]
    m = jnp.mean(t, axis=-1, keepdims=True)
    d = t - m
    v = jnp.mean(d * d, axis=-1, keepdims=True)
    o_ref[...] = d * lax.rsqrt(v + EPS) * g_ref[...] + b_ref[...]


def _normalize_table(table, gamma, beta):
    rows_blk = GENE_NUM // 10
    return pl.pallas_call(
        _ln_body,
        grid=(GENE_NUM // rows_blk,),
        in_specs=[
            pl.BlockSpec((rows_blk, D), lambda i: (i, 0)),
            pl.BlockSpec((1, D), lambda i: (0, 0)),
            pl.BlockSpec((1, D), lambda i: (0, 0)),
        ],
        out_specs=pl.BlockSpec((rows_blk, D), lambda i: (i, 0)),
        out_shape=jax.ShapeDtypeStruct((GENE_NUM, D), jnp.float32),
    )(table, gamma, beta)


@functools.partial(
    pl.kernel,
    mesh=plsc.VectorSubcoreMesh(core_axis_name="c", subcore_axis_name="s"),
    compiler_params=pltpu.CompilerParams(use_tc_tiling_on_sc=False),
    out_type=jax.ShapeDtypeStruct((BATCH, SEQ, D), jnp.float32),
    scratch_types=[
        pltpu.VMEM((NCH, CW), jnp.int32),
        pltpu.VMEM((2 * NB, CW, D), jnp.float32),
        pltpu.SemaphoreType.DMA,
        pltpu.SemaphoreType.DMA,
    ],
)
def _gather_kernel(table_hbm, idx_hbm, out_hbm, idx_v, rows_v, gsem, osem):
    wid = lax.axis_index("s") * NC + lax.axis_index("c")
    base_b = wid * BAT_W

    pltpu.sync_copy(idx_hbm.at[wid], idx_v)

    def gather_chunk(c, slot):
        pltpu.async_copy(table_hbm.at[idx_v.at[c]], rows_v.at[slot], gsem)

    def write_chunk(t, b, slot):
        k = base_b + 2 * t + (b // 2)
        s0 = (b % 2) * CW
        pltpu.async_copy(
            rows_v.at[slot], out_hbm.at[k, pl.ds(s0, CW)], osem)

    def drain(sem):
        pltpu.make_async_copy(
            rows_v.at[0], out_hbm.at[base_b, pl.ds(0, CW)], sem).wait()

    for b in range(NB):
        gather_chunk(b, b)

    def group_step(t, t2, par):
        off = par * NB
        if par == 1:
            for _ in range(NB):
                drain(osem)
        else:
            @pl.when(t2 > 0)
            def _():
                for _ in range(NB):
                    drain(osem)
        for _ in range(NB):
            drain(gsem)
        if par == 0:
            for b in range(NB):
                gather_chunk((t + 1) * NB + b, NB + b)
        else:
            @pl.when(t2 < NG // 2 - 1)
            def _():
                for b in range(NB):
                    gather_chunk((t + 1) * NB + b, b)
        for b in range(NB):
            write_chunk(t, b, off + b)

    def outer(t2, _):
        group_step(2 * t2, t2, 0)
        group_step(2 * t2 + 1, t2, 1)
        return 0

    lax.fori_loop(0, NG // 2, outer, 0)

    for _ in range(NB):
        drain(osem)


def kernel(x, table, gamma, beta):
    ntab = _normalize_table(table, gamma.reshape(1, D), beta.reshape(1, D))
    x3 = x.astype(jnp.int32).reshape(NW, NCH, CW)
    return _gather_kernel(ntab, x3)


# speedup vs baseline: 1.4034x; 1.4034x over previous
import functools

import jax
import jax.numpy as jnp
from jax import lax
from jax.experimental import pallas as pl
from jax.experimental.pallas import tpu as pltpu
from jax.experimental.pallas import tpu_sc as plsc

GENE_NUM = 100000
D = 64
BATCH = 4096
SEQ = 200
EPS = 1e-5

NC = 2
NS = 16
NW = NC * NS
CB = BATCH // 128
UNITS = SEQ * CB
U_W = UNITS // NW
NB = 4
NG = U_W // NB


def _ln_body(t_ref, g_ref, b_ref, o_ref):
    t = t_ref[...]
    m = jnp.mean(t, axis=-1, keepdims=True)
    d = t - m
    v = jnp.mean(d * d, axis=-1, keepdims=True)
    o_ref[...] = d * lax.rsqrt(v + EPS) * g_ref[...] + b_ref[...]


def _normalize_table(table, gamma, beta):
    rows_blk = GENE_NUM // 10
    return pl.pallas_call(
        _ln_body,
        grid=(GENE_NUM // rows_blk,),
        in_specs=[
            pl.BlockSpec((rows_blk, D), lambda i: (i, 0)),
            pl.BlockSpec((1, D), lambda i: (0, 0)),
            pl.BlockSpec((1, D), lambda i: (0, 0)),
        ],
        out_specs=pl.BlockSpec((rows_blk, D), lambda i: (i, 0)),
        out_shape=jax.ShapeDtypeStruct((GENE_NUM, D), jnp.float32),
    )(table, gamma, beta)


@functools.partial(
    pl.kernel,
    mesh=plsc.VectorSubcoreMesh(core_axis_name="c", subcore_axis_name="s"),
    compiler_params=pltpu.CompilerParams(use_tc_tiling_on_sc=False),
    out_type=jax.ShapeDtypeStruct((SEQ, BATCH, 128), jnp.float32),
    scratch_types=[
        pltpu.VMEM((U_W, 128), jnp.int32),
        pltpu.VMEM((2 * NB, 128, D), jnp.float32),
        pltpu.SemaphoreType.DMA,
        pltpu.SemaphoreType.DMA,
    ],
)
def _gather_kernel(table_hbm, idx_hbm, out_hbm, idx_v, rows_v, gsem, osem):
    wid = lax.axis_index("s") * NC + lax.axis_index("c")
    u0 = wid * U_W

    pltpu.sync_copy(idx_hbm.at[pl.ds(u0, U_W)], idx_v)

    def gather_chunk(k, slot):
        pltpu.async_copy(table_hbm.at[idx_v.at[k]], rows_v.at[slot], gsem)

    def write_chunk(k, slot):
        u = u0 + k
        s = u // CB
        c = u % CB
        pltpu.async_copy(
            rows_v.at[slot],
            out_hbm.at[s, pl.ds(c * 128, 128), pl.ds(0, D)], osem)

    def drain(sem):
        pltpu.make_async_copy(
            rows_v.at[0],
            out_hbm.at[0, pl.ds(0, 128), pl.ds(0, D)], sem).wait()

    for b in range(NB):
        gather_chunk(b, b)

    def group_step(t, t2, par):
        off = par * NB
        if par == 1:
            for _ in range(NB):
                drain(osem)
        else:
            @pl.when(t2 > 0)
            def _():
                for _ in range(NB):
                    drain(osem)
        for _ in range(NB):
            drain(gsem)
        if par == 0:
            for b in range(NB):
                gather_chunk((t + 1) * NB + b, NB + b)
        else:
            @pl.when(t2 < NG // 2 - 1)
            def _():
                for b in range(NB):
                    gather_chunk((t + 1) * NB + b, b)
        for b in range(NB):
            write_chunk(t * NB + b, off + b)

    def outer(t2, _):
        group_step(2 * t2, t2, 0)
        group_step(2 * t2 + 1, t2, 1)
        return 0

    lax.fori_loop(0, NG // 2, outer, 0)

    for _ in range(NB):
        drain(osem)


def _tr_body(i_ref, o_ref):
    o_ref[...] = jnp.transpose(i_ref[:, :, :D], (0, 2, 1))


def _transpose_pass(inter):
    return pl.pallas_call(
        _tr_body,
        grid=(SEQ,),
        in_specs=[pl.BlockSpec((1, BATCH, 128), lambda i: (i, 0, 0))],
        out_specs=pl.BlockSpec((1, D, BATCH), lambda i: (i, 0, 0)),
        out_shape=jax.ShapeDtypeStruct((SEQ, D, BATCH), jnp.float32),
    )(inter)


def kernel(x, table, gamma, beta):
    ntab = _normalize_table(table, gamma.reshape(1, D), beta.reshape(1, D))
    idx_t = x.astype(jnp.int32).T.reshape(UNITS, 128)
    inter = _gather_kernel(ntab, idx_t)
    out3 = _transpose_pass(inter)
    return jnp.transpose(out3, (2, 0, 1))


# speedup vs baseline: 1.4310x; 1.0197x over previous
import functools

import jax
import jax.numpy as jnp
from jax import lax
from jax.experimental import pallas as pl
from jax.experimental.pallas import tpu as pltpu
from jax.experimental.pallas import tpu_sc as plsc

GENE_NUM = 100000
D = 64
BATCH = 4096
SEQ = 200
EPS = 1e-5

NC = 2
NS = 16
NW = NC * NS
CB = BATCH // 128
HP = BATCH // 2
UNITS = SEQ * CB
U_W = UNITS // NW
NB = 4
NG = U_W // NB


def _ln_body(t_ref, g_ref, b_ref, o_ref):
    t = t_ref[...]
    m = jnp.mean(t, axis=-1, keepdims=True)
    d = t - m
    v = jnp.mean(d * d, axis=-1, keepdims=True)
    o_ref[...] = d * lax.rsqrt(v + EPS) * g_ref[...] + b_ref[...]


def _normalize_table(table, gamma, beta):
    rows_blk = GENE_NUM // 10
    return pl.pallas_call(
        _ln_body,
        grid=(GENE_NUM // rows_blk,),
        in_specs=[
            pl.BlockSpec((rows_blk, D), lambda i: (i, 0)),
            pl.BlockSpec((1, D), lambda i: (0, 0)),
            pl.BlockSpec((1, D), lambda i: (0, 0)),
        ],
        out_specs=pl.BlockSpec((rows_blk, D), lambda i: (i, 0)),
        out_shape=jax.ShapeDtypeStruct((GENE_NUM, D), jnp.float32),
    )(table, gamma, beta)


@functools.partial(
    pl.kernel,
    mesh=plsc.VectorSubcoreMesh(core_axis_name="c", subcore_axis_name="s"),
    compiler_params=pltpu.CompilerParams(use_tc_tiling_on_sc=False),
    out_type=jax.ShapeDtypeStruct((SEQ, HP, 128), jnp.float32),
    scratch_types=[
        pltpu.VMEM((U_W, 128), jnp.int32),
        pltpu.VMEM((2 * NB, 128, D), jnp.float32),
        pltpu.SemaphoreType.DMA,
        pltpu.SemaphoreType.DMA,
    ],
)
def _gather_kernel(table_hbm, idx_hbm, out_hbm, idx_v, rows_v, gsem, osem):
    wid = lax.axis_index("s") * NC + lax.axis_index("c")
    u0 = wid * U_W

    pltpu.sync_copy(idx_hbm.at[pl.ds(u0, U_W)], idx_v)

    def gather_chunk(k, slot):
        pltpu.async_copy(table_hbm.at[idx_v.at[k]], rows_v.at[slot], gsem)

    def write_chunk(k, slot):
        u = u0 + k
        s = u // CB
        c = u % CB
        half = c // NS
        p0 = (c % NS) * 128
        pltpu.async_copy(
            rows_v.at[slot],
            out_hbm.at[s, pl.ds(p0, 128), pl.ds(half * D, D)], osem)

    def drain(sem):
        pltpu.make_async_copy(
            rows_v.at[0],
            out_hbm.at[0, pl.ds(0, 128), pl.ds(0, D)], sem).wait()

    for b in range(NB):
        gather_chunk(b, b)

    def group_step(t, t2, par):
        off = par * NB
        if par == 1:
            for _ in range(NB):
                drain(osem)
        else:
            @pl.when(t2 > 0)
            def _():
                for _ in range(NB):
                    drain(osem)
        for _ in range(NB):
            drain(gsem)
        if par == 0:
            for b in range(NB):
                gather_chunk((t + 1) * NB + b, NB + b)
        else:
            @pl.when(t2 < NG // 2 - 1)
            def _():
                for b in range(NB):
                    gather_chunk((t + 1) * NB + b, b)
        for b in range(NB):
            write_chunk(t * NB + b, off + b)

    def outer(t2, _):
        group_step(2 * t2, t2, 0)
        group_step(2 * t2 + 1, t2, 1)
        return 0

    lax.fori_loop(0, NG // 2, outer, 0)

    for _ in range(NB):
        drain(osem)


def _tr_body(i_ref, o_ref):
    blk = i_ref[...]
    lo = jnp.transpose(blk[0, :, :D], (1, 0))
    hi = jnp.transpose(blk[0, :, D:], (1, 0))
    o_ref[...] = jnp.concatenate([lo, hi], axis=1)[None]


def _transpose_pass(inter):
    return pl.pallas_call(
        _tr_body,
        grid=(SEQ,),
        in_specs=[pl.BlockSpec((1, HP, 128), lambda i: (i, 0, 0))],
        out_specs=pl.BlockSpec((1, D, BATCH), lambda i: (i, 0, 0)),
        out_shape=jax.ShapeDtypeStruct((SEQ, D, BATCH), jnp.float32),
    )(inter)


def kernel(x, table, gamma, beta):
    ntab = _normalize_table(table, gamma.reshape(1, D), beta.reshape(1, D))
    idx_t = x.astype(jnp.int32).T.reshape(UNITS, 128)
    inter = _gather_kernel(ntab, idx_t)
    out3 = _transpose_pass(inter)
    return jnp.transpose(out3, (2, 0, 1))


# speedup vs baseline: 1.7265x; 1.2065x over previous
import functools

import jax
import jax.numpy as jnp
from jax import lax
from jax.experimental import pallas as pl
from jax.experimental.pallas import tpu as pltpu
from jax.experimental.pallas import tpu_sc as plsc

GENE_NUM = 100000
D = 64
BATCH = 4096
SEQ = 200
EPS = 1e-5

NC = 2
NS = 16
NW = NC * NS
CB = BATCH // 128
HP = BATCH // 2
UNITS = SEQ * CB
U_W = UNITS // NW
NB = 4
NG = U_W // NB


def _ln_body(t_ref, g_ref, b_ref, o_ref):
    t = t_ref[...]
    m = jnp.mean(t, axis=-1, keepdims=True)
    d = t - m
    v = jnp.mean(d * d, axis=-1, keepdims=True)
    o_ref[...] = d * lax.rsqrt(v + EPS) * g_ref[...] + b_ref[...]


def _normalize_table(table, gamma, beta):
    rows_blk = GENE_NUM // 10
    return pl.pallas_call(
        _ln_body,
        grid=(GENE_NUM // rows_blk,),
        in_specs=[
            pl.BlockSpec((rows_blk, D), lambda i: (i, 0)),
            pl.BlockSpec((1, D), lambda i: (0, 0)),
            pl.BlockSpec((1, D), lambda i: (0, 0)),
        ],
        out_specs=pl.BlockSpec((rows_blk, D), lambda i: (i, 0)),
        out_shape=jax.ShapeDtypeStruct((GENE_NUM, D), jnp.float32),
    )(table, gamma, beta)


@functools.partial(
    pl.kernel,
    mesh=plsc.VectorSubcoreMesh(core_axis_name="c", subcore_axis_name="s"),
    compiler_params=pltpu.CompilerParams(use_tc_tiling_on_sc=False),
    out_type=jax.ShapeDtypeStruct((SEQ, HP, 128), jnp.float32),
    scratch_types=[
        pltpu.VMEM((U_W, 128), jnp.int32),
        pltpu.VMEM((2 * NB, 128, D), jnp.float32),
        pltpu.SemaphoreType.DMA,
        pltpu.SemaphoreType.DMA,
    ],
)
def _gather_kernel(table_hbm, idx_hbm, out_hbm, idx_v, rows_v, gsem, osem):
    wid = lax.axis_index("s") * NC + lax.axis_index("c")
    u0 = wid * U_W

    pltpu.sync_copy(idx_hbm.at[pl.ds(u0, U_W)], idx_v)

    def gather_chunk(k, slot):
        pltpu.async_copy(table_hbm.at[idx_v.at[k]], rows_v.at[slot], gsem)

    def write_chunk(k, slot):
        u = u0 + k
        s = u // CB
        c = u % CB
        half = c // NS
        p0 = (c % NS) * 128
        pltpu.async_copy(
            rows_v.at[slot],
            out_hbm.at[s, pl.ds(p0, 128), pl.ds(half * D, D)], osem)

    def drain(sem):
        pltpu.make_async_copy(
            rows_v.at[0],
            out_hbm.at[0, pl.ds(0, 128), pl.ds(0, D)], sem).wait()

    for b in range(NB):
        gather_chunk(b, b)

    def group_step(t, t2, par):
        off = par * NB
        if par == 1:
            for _ in range(NB):
                drain(osem)
        else:
            @pl.when(t2 > 0)
            def _():
                for _ in range(NB):
                    drain(osem)
        for _ in range(NB):
            drain(gsem)
        if par == 0:
            for b in range(NB):
                gather_chunk((t + 1) * NB + b, NB + b)
        else:
            @pl.when(t2 < NG // 2 - 1)
            def _():
                for b in range(NB):
                    gather_chunk((t + 1) * NB + b, b)
        for b in range(NB):
            write_chunk(t * NB + b, off + b)

    def outer(t2, _):
        group_step(2 * t2, t2, 0)
        group_step(2 * t2 + 1, t2, 1)
        return 0

    lax.fori_loop(0, NG // 2, outer, 0)

    for _ in range(NB):
        drain(osem)


S_BLK = 8


def _tr_body(i_ref, o_ref):
    blk = i_ref[...]
    lo = jnp.transpose(blk[:, :, :D], (0, 2, 1))
    hi = jnp.transpose(blk[:, :, D:], (0, 2, 1))
    o_ref[...] = jnp.concatenate([lo, hi], axis=2)


def _transpose_pass(inter):
    return pl.pallas_call(
        _tr_body,
        grid=(SEQ // S_BLK,),
        in_specs=[pl.BlockSpec((S_BLK, HP, 128), lambda i: (i, 0, 0))],
        out_specs=pl.BlockSpec((S_BLK, D, BATCH), lambda i: (i, 0, 0)),
        out_shape=jax.ShapeDtypeStruct((SEQ, D, BATCH), jnp.float32),
    )(inter)


def kernel(x, table, gamma, beta):
    ntab = _normalize_table(table, gamma.reshape(1, D), beta.reshape(1, D))
    idx_t = x.astype(jnp.int32).T.reshape(UNITS, 128)
    inter = _gather_kernel(ntab, idx_t)
    out3 = _transpose_pass(inter)
    return jnp.transpose(out3, (2, 0, 1))


# speedup vs baseline: 1.7273x; 1.0005x over previous
import functools

import jax
import jax.numpy as jnp
from jax import lax
from jax.experimental import pallas as pl
from jax.experimental.pallas import tpu as pltpu
from jax.experimental.pallas import tpu_sc as plsc

GENE_NUM = 100000
D = 64
BATCH = 4096
SEQ = 200
EPS = 1e-5

NC = 2
NS = 16
NW = NC * NS
CB = BATCH // 128
HP = BATCH // 2
UNITS = SEQ * CB
U_W = UNITS // NW
NB = 4
NG = U_W // NB


def _ln_body(t_ref, g_ref, b_ref, o_ref):
    t = t_ref[...]
    m = jnp.mean(t, axis=-1, keepdims=True)
    d = t - m
    v = jnp.mean(d * d, axis=-1, keepdims=True)
    o_ref[...] = d * lax.rsqrt(v + EPS) * g_ref[...] + b_ref[...]


def _normalize_table(table, gamma, beta):
    rows_blk = GENE_NUM // 10
    return pl.pallas_call(
        _ln_body,
        grid=(GENE_NUM // rows_blk,),
        in_specs=[
            pl.BlockSpec((rows_blk, D), lambda i: (i, 0)),
            pl.BlockSpec((1, D), lambda i: (0, 0)),
            pl.BlockSpec((1, D), lambda i: (0, 0)),
        ],
        out_specs=pl.BlockSpec((rows_blk, D), lambda i: (i, 0)),
        out_shape=jax.ShapeDtypeStruct((GENE_NUM, D), jnp.float32),
    )(table, gamma, beta)


@functools.partial(
    pl.kernel,
    mesh=plsc.VectorSubcoreMesh(core_axis_name="c", subcore_axis_name="s"),
    compiler_params=pltpu.CompilerParams(use_tc_tiling_on_sc=False),
    out_type=jax.ShapeDtypeStruct((SEQ, HP, 128), jnp.float32),
    scratch_types=[
        pltpu.VMEM((U_W, 128), jnp.int32),
        pltpu.VMEM((2 * NB, 128, D), jnp.float32),
        pltpu.SemaphoreType.DMA,
        pltpu.SemaphoreType.DMA,
    ],
)
def _gather_kernel(table_hbm, idx_hbm, out_hbm, idx_v, rows_v, gsem, osem):
    wid = lax.axis_index("s") * NC + lax.axis_index("c")
    u0 = wid * U_W

    pltpu.sync_copy(idx_hbm.at[pl.ds(u0, U_W)], idx_v)

    def gather_chunk(k, slot):
        pltpu.async_copy(table_hbm.at[idx_v.at[k]], rows_v.at[slot], gsem)

    def write_chunk(k, slot):
        u = u0 + k
        s = u // CB
        c = u % CB
        half = c // NS
        p0 = (c % NS) * 128
        pltpu.async_copy(
            rows_v.at[slot],
            out_hbm.at[s, pl.ds(p0, 128), pl.ds(half * D, D)], osem)

    def drain(sem):
        pltpu.make_async_copy(
            rows_v.at[0],
            out_hbm.at[0, pl.ds(0, 128), pl.ds(0, D)], sem).wait()

    for b in range(NB):
        gather_chunk(b, b)

    def group_step(t, t2, par):
        off = par * NB
        if par == 1:
            for _ in range(NB):
                drain(osem)
        else:
            @pl.when(t2 > 0)
            def _():
                for _ in range(NB):
                    drain(osem)
        for _ in range(NB):
            drain(gsem)
        if par == 0:
            for b in range(NB):
                gather_chunk((t + 1) * NB + b, NB + b)
        else:
            @pl.when(t2 < NG // 2 - 1)
            def _():
                for b in range(NB):
                    gather_chunk((t + 1) * NB + b, b)
        for b in range(NB):
            write_chunk(t * NB + b, off + b)

    def outer(t2, _):
        group_step(2 * t2, t2, 0)
        group_step(2 * t2 + 1, t2, 1)
        return 0

    lax.fori_loop(0, NG // 2, outer, 0)

    for _ in range(NB):
        drain(osem)


S_BLK = 8


def _tr_body(i_ref, o_ref):
    blk = i_ref[...]
    o_ref[:, :, :HP] = jnp.transpose(blk[:, :, :D], (0, 2, 1))
    o_ref[:, :, HP:] = jnp.transpose(blk[:, :, D:], (0, 2, 1))


def _transpose_pass(inter):
    return pl.pallas_call(
        _tr_body,
        grid=(SEQ // S_BLK,),
        in_specs=[pl.BlockSpec((S_BLK, HP, 128), lambda i: (i, 0, 0))],
        out_specs=pl.BlockSpec((S_BLK, D, BATCH), lambda i: (i, 0, 0)),
        out_shape=jax.ShapeDtypeStruct((SEQ, D, BATCH), jnp.float32),
    )(inter)


def kernel(x, table, gamma, beta):
    ntab = _normalize_table(table, gamma.reshape(1, D), beta.reshape(1, D))
    idx_t = x.astype(jnp.int32).T.reshape(UNITS, 128)
    inter = _gather_kernel(ntab, idx_t)
    out3 = _transpose_pass(inter)
    return jnp.transpose(out3, (2, 0, 1))
